# baseline (speedup 1.0000x reference)
import functools

import jax
import jax.numpy as jnp
from jax import lax
from jax.experimental import pallas as pl
from jax.experimental.pallas import tpu as pltpu
from jax.experimental.pallas import tpu_sc as plsc

_D = 1024
_BATCH = 4
_SEQ = 8192
_NW = 32
_ROWS_PER_W = _SEQ // _NW
_R = 32
_CHUNK = _R * _D
_NCHUNK = _ROWS_PER_W // _R
_LANES = 16


def _pe_add_kernel(x_hbm, pe_hbm, out_hbm, pe_v, x_v, sem):
    cid = lax.axis_index("c")
    sid = lax.axis_index("s")
    wid = cid * 16 + sid
    row0 = wid * _ROWS_PER_W

    def chunk_body(c, _):
        pe_off = (row0 + c * _R) * _D
        pltpu.sync_copy(pe_hbm.at[pl.ds(pe_off, _CHUNK)], pe_v)

        def batch_body(b, _):
            x_off = b * (_SEQ * _D) + pe_off
            pltpu.sync_copy(x_hbm.at[pl.ds(x_off, _CHUNK)], x_v)

            def add_body(i, _):
                sl = pl.ds(i * _LANES, _LANES)
                x_v[sl] = x_v[sl] + pe_v[sl]
                return 0

            lax.fori_loop(0, _CHUNK // _LANES, add_body, 0)
            pltpu.sync_copy(x_v, out_hbm.at[pl.ds(x_off, _CHUNK)])
            return 0

        lax.fori_loop(0, _BATCH, batch_body, 0)
        return 0

    lax.fori_loop(0, _NCHUNK, chunk_body, 0)


@jax.jit
def kernel(x, pe_weight):
    x_flat = x.reshape(-1)
    pe_flat = pe_weight.reshape(-1)
    mesh = plsc.VectorSubcoreMesh(core_axis_name="c", subcore_axis_name="s")
    run = functools.partial(
        pl.kernel,
        mesh=mesh,
        out_type=jax.ShapeDtypeStruct((_BATCH * _SEQ * _D,), jnp.float32),
        scratch_types=[
            pltpu.VMEM((_CHUNK,), jnp.float32),
            pltpu.VMEM((_CHUNK,), jnp.float32),
            pltpu.SemaphoreType.DMA,
        ],
    )(_pe_add_kernel)
    out = run(x_flat, pe_flat)
    return out.reshape(x.shape)


# speedup vs baseline: 1.5512x; 1.5512x over previous
import functools

import jax
import jax.numpy as jnp
from jax import lax
from jax.experimental import pallas as pl
from jax.experimental.pallas import tpu as pltpu
from jax.experimental.pallas import tpu_sc as plsc

_D = 1024
_BATCH = 4
_SEQ = 8192
_NW = 32
_ROWS_PER_W = _SEQ // _NW
_R = 16
_CHUNK = _R * _D
_NCHUNK = _ROWS_PER_W // _R
_LANES = 16
_UNROLL = 8


def _pe_add_kernel(x_hbm, pe_hbm, out_hbm, pe_v, x_v, pe_sem, in_sem, out_sem):
    cid = lax.axis_index("c")
    sid = lax.axis_index("s")
    wid = cid * 16 + sid
    row0 = wid * _ROWS_PER_W

    def pe_off(c):
        return (row0 + c * _R) * _D

    def x_off(c, b):
        return b * (_SEQ * _D) + pe_off(c)

    def start_pe(c, buf):
        pltpu.async_copy(pe_hbm.at[pl.ds(pe_off(c), _CHUNK)], pe_v.at[buf],
                         pe_sem)

    def start_in(c, b, buf):
        pltpu.async_copy(x_hbm.at[pl.ds(x_off(c, b), _CHUNK)], x_v.at[buf],
                         in_sem)

    def wait_pe():
        pltpu.make_async_copy(pe_hbm.at[pl.ds(0, _CHUNK)], pe_v.at[0],
                              pe_sem).wait()

    def wait_in():
        pltpu.make_async_copy(x_hbm.at[pl.ds(0, _CHUNK)], x_v.at[0],
                              in_sem).wait()

    def wait_out():
        pltpu.make_async_copy(x_v.at[0], out_hbm.at[pl.ds(0, _CHUNK)],
                              out_sem).wait()

    def add_chunk(xb, pb):
        def add_body(i, _):
            base = i * (_LANES * _UNROLL)
            for u in range(_UNROLL):
                sl = pl.ds(base + u * _LANES, _LANES)
                plsc.addupdate(x_v.at[xb, sl], pe_v[pb, sl])
            return 0

        lax.fori_loop(0, _CHUNK // (_LANES * _UNROLL), add_body, 0)

    start_pe(0, 0)
    start_in(0, 0, 0)

    def chunk_pair(c2, _):
        for cc in (0, 1):
            c = 2 * c2 + cc
            wait_pe()
            if cc == 0:
                start_pe(c + 1, 1)
            else:
                @pl.when(c2 != _NCHUNK // 2 - 1)
                def _():
                    start_pe(c + 1, 0)
            for b in range(_BATCH):
                if cc == 0 and b == 0:
                    @pl.when(c2 != 0)
                    def _():
                        wait_out()
                else:
                    wait_out()
                if b < _BATCH - 1:
                    start_in(c, b + 1, (b + 1) % 2)
                elif cc == 0:
                    start_in(c + 1, 0, 0)
                else:
                    @pl.when(c2 != _NCHUNK // 2 - 1)
                    def _():
                        start_in(c + 1, 0, 0)
                wait_in()
                add_chunk(b % 2, cc)
                pltpu.async_copy(x_v.at[b % 2],
                                 out_hbm.at[pl.ds(x_off(c, b), _CHUNK)],
                                 out_sem)
        return 0

    lax.fori_loop(0, _NCHUNK // 2, chunk_pair, 0)
    wait_out()


@jax.jit
def kernel(x, pe_weight):
    x_flat = x.reshape(-1)
    pe_flat = pe_weight.reshape(-1)
    mesh = plsc.VectorSubcoreMesh(core_axis_name="c", subcore_axis_name="s")
    run = functools.partial(
        pl.kernel,
        mesh=mesh,
        out_type=jax.ShapeDtypeStruct((_BATCH * _SEQ * _D,), jnp.float32),
        scratch_types=[
            pltpu.VMEM((2, _CHUNK), jnp.float32),
            pltpu.VMEM((2, _CHUNK), jnp.float32),
            pltpu.SemaphoreType.DMA,
            pltpu.SemaphoreType.DMA,
            pltpu.SemaphoreType.DMA,
        ],
    )(_pe_add_kernel)
    out = run(x_flat, pe_flat)
    return out.reshape(x.shape)


# speedup vs baseline: 2.3439x; 1.5110x over previous
import functools

import jax
import jax.numpy as jnp
from jax import lax
from jax.experimental import pallas as pl
from jax.experimental.pallas import tpu as pltpu
from jax.experimental.pallas import tpu_sc as plsc

_D = 1024
_BATCH = 4
_SEQ = 8192
_NW = 32
_ROWS_PER_W = _SEQ // _NW
_R = 16
_NCHUNK = _ROWS_PER_W // _R
_LANES = 16
_DSLICES = _D // _LANES


def _pe_add_kernel(x_hbm, pe_hbm, out_hbm, pe_v, x_v, pe_sem, in_sem, out_sem):
    cid = lax.axis_index("c")
    sid = lax.axis_index("s")
    wid = cid * 16 + sid
    row0 = wid * _ROWS_PER_W

    def start_pe(c, buf):
        pltpu.async_copy(pe_hbm.at[pl.ds(row0 + c * _R, _R)], pe_v.at[buf],
                         pe_sem)

    def start_in(c, b, buf):
        pltpu.async_copy(x_hbm.at[b, pl.ds(row0 + c * _R, _R)], x_v.at[buf],
                         in_sem)

    def wait_pe():
        pltpu.make_async_copy(pe_hbm.at[pl.ds(0, _R)], pe_v.at[0],
                              pe_sem).wait()

    def wait_in():
        pltpu.make_async_copy(pe_hbm.at[pl.ds(0, _R)], x_v.at[0],
                              in_sem).wait()

    def wait_out():
        pltpu.make_async_copy(x_v.at[0], out_hbm.at[0, pl.ds(0, _R)],
                              out_sem).wait()

    def add_chunk(xb, pb):
        def add_body(r, _):
            for u in range(_DSLICES):
                sl = pl.ds(u * _LANES, _LANES)
                plsc.addupdate(x_v.at[xb, r, sl], pe_v[pb, r, sl])
            return 0

        lax.fori_loop(0, _R, add_body, 0)

    start_pe(0, 0)
    start_in(0, 0, 0)

    def chunk_pair(c2, _):
        for cc in (0, 1):
            c = 2 * c2 + cc
            wait_pe()
            if cc == 0:
                start_pe(c + 1, 1)
            else:
                @pl.when(c2 != _NCHUNK // 2 - 1)
                def _():
                    start_pe(c + 1, 0)
            for b in range(_BATCH):
                if cc == 0 and b == 0:
                    @pl.when(c2 != 0)
                    def _():
                        wait_out()
                else:
                    wait_out()
                if b < _BATCH - 1:
                    start_in(c, b + 1, (b + 1) % 2)
                elif cc == 0:
                    start_in(c + 1, 0, 0)
                else:
                    @pl.when(c2 != _NCHUNK // 2 - 1)
                    def _():
                        start_in(c + 1, 0, 0)
                wait_in()
                add_chunk(b % 2, cc)
                pltpu.async_copy(x_v.at[b % 2],
                                 out_hbm.at[b, pl.ds(row0 + c * _R, _R)],
                                 out_sem)
        return 0

    lax.fori_loop(0, _NCHUNK // 2, chunk_pair, 0)
    wait_out()


@jax.jit
def kernel(x, pe_weight):
    mesh = plsc.VectorSubcoreMesh(core_axis_name="c", subcore_axis_name="s")
    run = functools.partial(
        pl.kernel,
        mesh=mesh,
        out_type=jax.ShapeDtypeStruct((_BATCH, _SEQ, _D), jnp.float32),
        scratch_types=[
            pltpu.VMEM((2, _R, _D), jnp.float32),
            pltpu.VMEM((2, _R, _D), jnp.float32),
            pltpu.SemaphoreType.DMA,
            pltpu.SemaphoreType.DMA,
            pltpu.SemaphoreType.DMA,
        ],
    )(_pe_add_kernel)
    return run(x, pe_weight)


# speedup vs baseline: 2.7329x; 1.1659x over previous
import functools

import jax
import jax.numpy as jnp
from jax import lax
from jax.experimental import pallas as pl
from jax.experimental.pallas import tpu as pltpu
from jax.experimental.pallas import tpu_sc as plsc

_D = 1024
_BATCH = 4
_SEQ = 8192
_NW = 32
_ROWS_PER_W = _SEQ // _NW
_R = 16
_NCHUNK = _ROWS_PER_W // _R
_LANES = 16
_DSLICES = _D // _LANES


def _pe_add_kernel(x_hbm, pe_hbm, out_hbm, pe_v, x_v, pe_sem, in_sem, out_sem):
    cid = lax.axis_index("c")
    sid = lax.axis_index("s")
    wid = cid * 16 + sid
    row0 = wid * _ROWS_PER_W

    def start_pe(c, buf):
        pltpu.async_copy(pe_hbm.at[pl.ds(row0 + c * _R, _R)], pe_v.at[buf],
                         pe_sem)

    def start_in(c, b, buf):
        pltpu.async_copy(x_hbm.at[b, pl.ds(row0 + c * _R, _R)], x_v.at[buf],
                         in_sem)

    def wait_pe():
        pltpu.make_async_copy(pe_hbm.at[pl.ds(0, _R)], pe_v.at[0],
                              pe_sem).wait()

    def wait_in():
        pltpu.make_async_copy(pe_hbm.at[pl.ds(0, _R)], x_v.at[0],
                              in_sem).wait()

    def wait_out():
        pltpu.make_async_copy(x_v.at[0], out_hbm.at[0, pl.ds(0, _R)],
                              out_sem).wait()

    def add_chunk(xb, pb):
        def add_body(r, _):
            for u in range(_DSLICES):
                sl = pl.ds(u * _LANES, _LANES)
                plsc.addupdate(x_v.at[xb, r, sl], pe_v[pb, r, sl])
            return 0

        lax.fori_loop(0, _R, add_body, 0)

    start_pe(0, 0)
    start_in(0, 0, 0)

    def chunk_pair(c2, _):
        for cc in (0, 1):
            c = 2 * c2 + cc
            wait_pe()
            if cc == 0:
                start_pe(c + 1, 1)
            else:
                @pl.when(c2 != _NCHUNK // 2 - 1)
                def _():
                    start_pe(c + 1, 0)
            for b in range(_BATCH):
                if cc == 0 and b < _BATCH - 1:
                    @pl.when(c2 != 0)
                    def _():
                        wait_out()
                else:
                    wait_out()
                if b < _BATCH - 1:
                    start_in(c, b + 1, b + 1)
                elif cc == 0:
                    start_in(c + 1, 0, 0)
                else:
                    @pl.when(c2 != _NCHUNK // 2 - 1)
                    def _():
                        start_in(c + 1, 0, 0)
                wait_in()
                add_chunk(b, cc)
                pltpu.async_copy(x_v.at[b],
                                 out_hbm.at[b, pl.ds(row0 + c * _R, _R)],
                                 out_sem)
        return 0

    lax.fori_loop(0, _NCHUNK // 2, chunk_pair, 0)
    wait_out()
    wait_out()
    wait_out()


@jax.jit
def kernel(x, pe_weight):
    mesh = plsc.VectorSubcoreMesh(core_axis_name="c", subcore_axis_name="s")
    run = functools.partial(
        pl.kernel,
        mesh=mesh,
        out_type=jax.ShapeDtypeStruct((_BATCH, _SEQ, _D), jnp.float32),
        scratch_types=[
            pltpu.VMEM((2, _R, _D), jnp.float32),
            pltpu.VMEM((_BATCH, _R, _D), jnp.float32),
            pltpu.SemaphoreType.DMA,
            pltpu.SemaphoreType.DMA,
            pltpu.SemaphoreType.DMA,
        ],
    )(_pe_add_kernel)
    return run(x, pe_weight)


# speedup vs baseline: 5.1952x; 1.9010x over previous
import functools

import jax
import jax.numpy as jnp
from jax import lax
from jax.experimental import pallas as pl
from jax.experimental.pallas import tpu as pltpu
from jax.experimental.pallas import tpu_sc as plsc

_D = 1024
_BATCH = 4
_SEQ = 8192
_NW = 32
_ROWS_PER_W = _SEQ // _NW
_R = 16
_NCHUNK = _ROWS_PER_W // _R
_LANES = 16
_DSLICES = _D // _LANES


def _pe_add_kernel(x_hbm, pe_hbm, out_hbm, pe_v, x_v, pe_sem, in_sem, out_sem):
    cid = lax.axis_index("c")
    sid = lax.axis_index("s")
    wid = cid * 16 + sid
    row0 = wid * _ROWS_PER_W

    def start_pe(c, buf):
        pltpu.async_copy(pe_hbm.at[pl.ds(row0 + c * _R, _R)], pe_v.at[buf],
                         pe_sem)

    def start_in(c, b, buf):
        pltpu.async_copy(x_hbm.at[b, pl.ds(row0 + c * _R, _R)], x_v.at[buf],
                         in_sem)

    def wait_pe():
        pltpu.make_async_copy(pe_hbm.at[pl.ds(0, _R)], pe_v.at[0],
                              pe_sem).wait()

    def wait_in():
        pltpu.make_async_copy(pe_hbm.at[pl.ds(0, _R)], x_v.at[0],
                              in_sem).wait()

    def wait_out():
        pltpu.make_async_copy(x_v.at[0], out_hbm.at[0, pl.ds(0, _R)],
                              out_sem).wait()

    def add_chunk(xb, pb):
        def add_body(r, _):
            for g in range(0, _DSLICES, 8):
                vals = [pe_v[pb, r, pl.ds((g + k) * _LANES, _LANES)]
                        for k in range(8)]
                for k in range(8):
                    plsc.addupdate(
                        x_v.at[xb, r, pl.ds((g + k) * _LANES, _LANES)],
                        vals[k])
            return 0

        lax.fori_loop(0, _R, add_body, 0)

    start_pe(0, 0)
    start_in(0, 0, 0)

    def chunk_pair(c2, _):
        for cc in (0, 1):
            c = 2 * c2 + cc
            wait_pe()
            if cc == 0:
                start_pe(c + 1, 1)
            else:
                @pl.when(c2 != _NCHUNK // 2 - 1)
                def _():
                    start_pe(c + 1, 0)
            for b in range(_BATCH):
                if cc == 0 and b < _BATCH - 1:
                    @pl.when(c2 != 0)
                    def _():
                        wait_out()
                else:
                    wait_out()
                if b < _BATCH - 1:
                    start_in(c, b + 1, b + 1)
                elif cc == 0:
                    start_in(c + 1, 0, 0)
                else:
                    @pl.when(c2 != _NCHUNK // 2 - 1)
                    def _():
                        start_in(c + 1, 0, 0)
                wait_in()
                add_chunk(b, cc)
                pltpu.async_copy(x_v.at[b],
                                 out_hbm.at[b, pl.ds(row0 + c * _R, _R)],
                                 out_sem)
        return 0

    lax.fori_loop(0, _NCHUNK // 2, chunk_pair, 0)
    wait_out()
    wait_out()
    wait_out()


@jax.jit
def kernel(x, pe_weight):
    mesh = plsc.VectorSubcoreMesh(core_axis_name="c", subcore_axis_name="s")
    run = functools.partial(
        pl.kernel,
        mesh=mesh,
        out_type=jax.ShapeDtypeStruct((_BATCH, _SEQ, _D), jnp.float32),
        scratch_types=[
            pltpu.VMEM((2, _R, _D), jnp.float32),
            pltpu.VMEM((_BATCH, _R, _D), jnp.float32),
            pltpu.SemaphoreType.DMA,
            pltpu.SemaphoreType.DMA,
            pltpu.SemaphoreType.DMA,
        ],
    )(_pe_add_kernel)
    return run(x, pe_weight)
